# baseline (speedup 1.0000x reference)
import functools

import jax
import jax.numpy as jnp
from jax import lax
from jax.experimental import pallas as pl
from jax.experimental.pallas import tpu as pltpu
from jax.experimental.pallas import tpu_sc as plsc

N = 100000
VA = 1000000
VR = 100000
D_ATOM = 64
D_RES = 32
D_NUM = 16
D_OUT = D_ATOM + D_RES + D_NUM

NC = 2
NS = 16
NW = NC * NS

CHUNK = 256
NCHUNK = 13
BPW = CHUNK * NCHUNK
NP = NW * BPW

VB = 512


def _relayout_atom(wt):
    def body(in_ref, out_ref):
        y = in_ref[...].T
        out_ref[...] = jnp.concatenate([y, y], axis=1)

    grid = pl.cdiv(VA, VB)
    return pl.pallas_call(
        body,
        grid=(grid,),
        in_specs=[pl.BlockSpec((64, VB), lambda b: (0, b))],
        out_specs=pl.BlockSpec((VB, 128), lambda b: (b, 0)),
        out_shape=jax.ShapeDtypeStruct((VA, 128), jnp.float32),
    )(wt)


def _relayout_res(wt):
    def body(in_ref, out_ref):
        y = in_ref[...].T
        out_ref[...] = jnp.concatenate([y, y, y, y], axis=1)

    grid = pl.cdiv(VR, VB)
    return pl.pallas_call(
        body,
        grid=(grid,),
        in_specs=[pl.BlockSpec((32, VB), lambda b: (0, b))],
        out_specs=pl.BlockSpec((VB, 128), lambda b: (b, 0)),
        out_shape=jax.ShapeDtypeStruct((VR, 128), jnp.float32),
    )(wt)


def _make_sc_kernel():
    mesh = plsc.VectorSubcoreMesh(core_axis_name="c", subcore_axis_name="s")

    @functools.partial(
        pl.kernel,
        mesh=mesh,
        out_type=jax.ShapeDtypeStruct((NP, 128), jnp.float32),
        compiler_params=pltpu.CompilerParams(needs_layout_passes=False),
        scratch_types=[
            pltpu.VMEM((CHUNK,), jnp.int32),
            pltpu.VMEM((CHUNK,), jnp.int32),
            pltpu.VMEM((CHUNK, 128), jnp.float32),
            pltpu.VMEM((CHUNK, 128), jnp.float32),
            pltpu.VMEM((16, CHUNK), jnp.float32),
            pltpu.SemaphoreType.DMA,
            pltpu.SemaphoreType.DMA,
        ],
    )
    def emb_kernel(idx_a, idx_r, ch_t, wa, wr, out,
                   idxa_v, idxr_v, stage, gr, chv, sem_a, sem_r):
        wid = lax.axis_index("s") * NC + lax.axis_index("c")
        base = wid * BPW

        def chunk_body(ci, carry):
            start = base + ci * CHUNK
            pltpu.sync_copy(idx_a.at[pl.ds(start, CHUNK)], idxa_v)
            pltpu.sync_copy(idx_r.at[pl.ds(start, CHUNK)], idxr_v)
            cpa = pltpu.async_copy(wa.at[idxa_v], stage, sem_a)
            cpr = pltpu.async_copy(wr.at[idxr_v], gr, sem_r)
            pltpu.sync_copy(ch_t.at[:, pl.ds(start, CHUNK)], chv)
            cpa.wait()
            cpr.wait()

            def node_body(j, carry2):
                for m in range(2):
                    stage[j, pl.ds(64 + 16 * m, 16)] = gr[j, pl.ds(16 * m, 16)]
                cvals = plsc.load_gather(
                    chv,
                    [lax.iota(jnp.int32, 16), jnp.full((16,), j, jnp.int32)])
                stage[j, pl.ds(96, 16)] = cvals
                return carry2

            lax.fori_loop(0, CHUNK, node_body, 0)
            pltpu.sync_copy(stage, out.at[pl.ds(start, CHUNK)])
            return carry

        lax.fori_loop(0, NCHUNK, chunk_body, 0)

    return emb_kernel


_SC_EMB = _make_sc_kernel()


def kernel(atom_type, residue_type, charge, W_atom, W_res):
    idx_a = atom_type.reshape(-1).astype(jnp.int32)
    idx_r = residue_type.reshape(-1).astype(jnp.int32)
    pad = (0, NP - N)
    idx_a_p = jnp.pad(idx_a, pad)
    idx_r_p = jnp.pad(idx_r, pad)
    ch_t = jnp.pad(charge.T, ((0, 0), pad))
    wa = _relayout_atom(W_atom.T)
    wr = _relayout_res(W_res.T)
    out = _SC_EMB(idx_a_p, idx_r_p, ch_t, wa, wr)
    return out[:N, :D_OUT]


# speedup vs baseline: 3.1088x; 3.1088x over previous
import functools

import jax
import jax.numpy as jnp
from jax import lax
from jax.experimental import pallas as pl
from jax.experimental.pallas import tpu as pltpu
from jax.experimental.pallas import tpu_sc as plsc

N = 100000
VA = 1000000
VR = 100000
D_ATOM = 64
D_RES = 32
D_OUT = 112

NC = 2
NS = 16
NW = NC * NS

CHUNK = 224
NCHUNK = 14
BPW = CHUNK * NCHUNK
NP = NW * BPW

VB = 8192


def _relayout_atom(wt):
    def body(in_ref, out_ref):
        y = in_ref[...].T
        out_ref[...] = jnp.concatenate([y, y], axis=1)

    grid = pl.cdiv(VA, VB)
    return pl.pallas_call(
        body,
        grid=(grid,),
        in_specs=[pl.BlockSpec((64, VB), lambda b: (0, b))],
        out_specs=pl.BlockSpec((VB, 128), lambda b: (b, 0)),
        out_shape=jax.ShapeDtypeStruct((VA, 128), jnp.float32),
        compiler_params=pltpu.CompilerParams(
            dimension_semantics=("arbitrary",)),
    )(wt)


def _relayout_res(wt):
    def body(in_ref, out_ref):
        y = in_ref[...].T
        out_ref[...] = jnp.concatenate([y, y, y, y], axis=1)

    grid = pl.cdiv(VR, VB)
    return pl.pallas_call(
        body,
        grid=(grid,),
        in_specs=[pl.BlockSpec((32, VB), lambda b: (0, b))],
        out_specs=pl.BlockSpec((VB, 128), lambda b: (b, 0)),
        out_shape=jax.ShapeDtypeStruct((VR, 128), jnp.float32),
        compiler_params=pltpu.CompilerParams(
            dimension_semantics=("arbitrary",)),
    )(wt)


def _make_sc_kernel():
    mesh = plsc.VectorSubcoreMesh(core_axis_name="c", subcore_axis_name="s")

    @functools.partial(
        pl.kernel,
        mesh=mesh,
        out_type=jax.ShapeDtypeStruct((NP, 128), jnp.float32),
        compiler_params=pltpu.CompilerParams(needs_layout_passes=False),
        scratch_types=[
            pltpu.VMEM((CHUNK,), jnp.int32),
            pltpu.VMEM((CHUNK,), jnp.int32),
            pltpu.VMEM((CHUNK,), jnp.int32),
            pltpu.VMEM((CHUNK,), jnp.int32),
            pltpu.VMEM((CHUNK, 128), jnp.float32),
            pltpu.VMEM((CHUNK, 128), jnp.float32),
            pltpu.VMEM((CHUNK, 128), jnp.float32),
            pltpu.VMEM((CHUNK, 128), jnp.float32),
            pltpu.SemaphoreType.DMA,
            pltpu.SemaphoreType.DMA,
            pltpu.SemaphoreType.DMA,
            pltpu.SemaphoreType.DMA,
        ],
    )
    def emb_kernel(idx_a, idx_r, wa, wr, out,
                   idxa0, idxa1, idxr0, idxr1,
                   stage0, stage1, gr0, gr1, sa0, sa1, sr0, sr1):
        wid = lax.axis_index("s") * NC + lax.axis_index("c")
        base = wid * BPW
        idxa = (idxa0, idxa1)
        idxr = (idxr0, idxr1)
        stage = (stage0, stage1)
        gr = (gr0, gr1)
        sem_a = (sa0, sa1)
        sem_r = (sr0, sr1)

        def fetch(ci, b):
            start = base + ci * CHUNK
            pltpu.sync_copy(idx_a.at[pl.ds(start, CHUNK)], idxa[b])
            pltpu.sync_copy(idx_r.at[pl.ds(start, CHUNK)], idxr[b])
            pltpu.async_copy(wa.at[idxa[b]], stage[b], sem_a[b])
            pltpu.async_copy(wr.at[idxr[b]], gr[b], sem_r[b])

        fetch(0, 0)

        def pair_body(g, carry):
            for b in (0, 1):
                ci = 2 * g + b

                @pl.when(ci + 1 < NCHUNK)
                def _prefetch():
                    fetch(ci + 1, 1 - b)

                pltpu.make_async_copy(
                    wa.at[idxa[b]], stage[b], sem_a[b]).wait()
                pltpu.make_async_copy(
                    wr.at[idxr[b]], gr[b], sem_r[b]).wait()

                def node_body(i, carry2):
                    for l in range(8):
                        j = i * 8 + l
                        for m in range(2):
                            stage[b][j, pl.ds(64 + 16 * m, 16)] = (
                                gr[b][j, pl.ds(16 * m, 16)])
                    return carry2

                lax.fori_loop(0, CHUNK // 8, node_body, 0)
                pltpu.sync_copy(stage[b],
                                out.at[pl.ds(base + ci * CHUNK, CHUNK)])
            return carry

        lax.fori_loop(0, NCHUNK // 2, pair_body, 0)

    return emb_kernel


_SC_EMB = _make_sc_kernel()


def kernel(atom_type, residue_type, charge, W_atom, W_res):
    idx_a = atom_type.reshape(-1).astype(jnp.int32)
    idx_r = residue_type.reshape(-1).astype(jnp.int32)
    pad = (0, NP - N)
    idx_a_p = jnp.pad(idx_a, pad)
    idx_r_p = jnp.pad(idx_r, pad)
    wa = _relayout_atom(W_atom.T)
    wr = _relayout_res(W_res.T)
    out = _SC_EMB(idx_a_p, idx_r_p, wa, wr)
    return jnp.concatenate([out[:N, :96], charge], axis=-1)
